# baseline (speedup 1.0000x reference)
import functools

import jax
import jax.numpy as jnp
from jax import lax
from jax.experimental import pallas as pl
from jax.experimental.pallas import tpu as pltpu
from jax.experimental.pallas import tpu_sc as plsc

B = 16384
L = 200
D = 32
V = 1_000_000
NC = 2
NS = 16
NW = NC * NS

BLK = 128
NBLK = V // BLK
TAIL = V - NBLK * BLK
VPAD = (NBLK + 1) * BLK
OUT_ROWS = VPAD * D // BLK
BASE_BLKS = NBLK // NW
EXTRA = NBLK - BASE_BLKS * NW

ROWS_PER_W = B // NW
G = 8
CHUNKS = ROWS_PER_W // G
IDX_SPLITS = ((0, 104), (104, 96))
INV_L = 1.0 / L

_mesh = plsc.VectorSubcoreMesh(core_axis_name="c", subcore_axis_name="s")


@functools.partial(
    pl.kernel,
    out_type=jax.ShapeDtypeStruct((VPAD * D,), jnp.float32),
    mesh=_mesh,
    compiler_params=pltpu.CompilerParams(use_tc_tiling_on_sc=True,
                                         needs_layout_passes=False),
    scratch_types=[
        pltpu.VMEM((D, BLK), jnp.float32),
        pltpu.VMEM((D, BLK), jnp.float32),
        pltpu.VMEM((D * BLK,), jnp.float32),
        pltpu.VMEM((D * BLK,), jnp.float32),
        pltpu.SemaphoreType.DMA,
        pltpu.SemaphoreType.DMA,
    ],
)
def _transpose_table(tab_t, tail, out, in0, in1, st0, st1, isem, osem):
    wid = lax.axis_index("s") * NC + lax.axis_index("c")
    nblk = jnp.where(wid < EXTRA, BASE_BLKS + 1, BASE_BLKS)
    start = wid * BASE_BLKS + jnp.minimum(wid, EXTRA)
    in_bufs = (in0, in1)
    st_bufs = (st0, st1)

    f_iota = lax.iota(jnp.int32, 16)
    p_vec = f_iota * D

    def fetch(i, buf):
        c = pl.multiple_of((start + i) * BLK, BLK)
        pltpu.async_copy(tab_t.at[:, pl.ds(c, BLK)], in_bufs[buf], isem)

    def wait_fetch(buf):
        pltpu.make_async_copy(tab_t.at[:, pl.ds(0, BLK)], in_bufs[buf],
                              isem).wait()

    def transpose_block(i, buf):
        in_v = in_bufs[buf]
        st_v = st_bufs[buf]

        @plsc.parallel_loop(0, BLK, step=16, unroll=4)
        def _(v0):
            base = pl.multiple_of(v0, 16) * D
            for f in range(D):
                x = in_v[f, pl.ds(v0, 16)]
                plsc.store_scatter(st_v, [p_vec + (base + f)], x)

        o = pl.multiple_of((start + i) * D * BLK, D * BLK)
        pltpu.async_copy(st_v, out.at[pl.ds(o, D * BLK)], osem)

    def wait_out(buf):
        pltpu.make_async_copy(out.at[pl.ds(0, D * BLK)], st_bufs[buf],
                              osem).wait()

    fetch(0, 0)

    def pair_body(i, _):
        i0 = i * 2
        i1 = i0 + 1

        @pl.when(i1 < nblk)
        def _():
            fetch(i1, 1)

        @pl.when(i0 < nblk)
        def _():
            wait_fetch(0)
            transpose_block(i0, 0)

        @pl.when(i0 + 2 < nblk)
        def _():
            fetch(i0 + 2, 0)

        @pl.when(i1 < nblk)
        def _():
            wait_fetch(1)
            transpose_block(i1, 1)
            wait_out(1)

        @pl.when(i0 < nblk)
        def _():
            wait_out(0)

        return ()

    lax.fori_loop(0, (BASE_BLKS + 2) // 2, pair_body, ())

    @pl.when(wid == 0)
    def _():
        pltpu.sync_copy(tail, in0.at[pl.ds(0, 16)])
        for r in range(16):
            for c0 in range(0, BLK, 16):
                st0[pl.ds(r * BLK + c0, 16)] = in0[r, pl.ds(c0, 16)]
        pltpu.sync_copy(st0.at[pl.ds(0, TAIL * D)],
                        out.at[pl.ds(NBLK * BLK * D, TAIL * D)])


@functools.partial(
    pl.kernel,
    out_type=jax.ShapeDtypeStruct((B, D), jnp.float32),
    mesh=_mesh,
    compiler_params=pltpu.CompilerParams(use_tc_tiling_on_sc=False),
    scratch_types=[
        pltpu.VMEM((2, G, L), jnp.int32),
        pltpu.VMEM((G * L, D), jnp.float32),
        pltpu.VMEM((G * L, D), jnp.float32),
        pltpu.VMEM((G, D), jnp.float32),
        pltpu.SemaphoreType.DMA,
        pltpu.SemaphoreType.DMA,
    ],
)
def _pooled_gather(table_hbm, idx_hbm, out_hbm, idx_v, rows0_v, rows1_v,
                   out_v, sem0, sem1):
    wid = lax.axis_index("s") * NC + lax.axis_index("c")
    row_base = wid * ROWS_PER_W
    rows_bufs = (rows0_v, rows1_v)
    sems = (sem0, sem1)

    def start(c, buf):
        out_base = pl.multiple_of(row_base + c * G, G)
        pltpu.sync_copy(idx_hbm.at[pl.ds(out_base, G)], idx_v.at[buf])
        for g in range(G):
            for off, size in IDX_SPLITS:
                pltpu.async_copy(
                    table_hbm.at[idx_v.at[buf, g, pl.ds(off, size)]],
                    rows_bufs[buf].at[pl.ds(g * L + off, size)],
                    sems[buf],
                )

    def drain(buf):
        pltpu.make_async_copy(
            table_hbm.at[pl.ds(0, G * L)], rows_bufs[buf], sems[buf]
        ).wait()

    def reduce_store(c, buf):
        rows_v = rows_bufs[buf]
        out_base = pl.multiple_of(row_base + c * G, G)
        for g in range(G):
            def red_body(i, accs):
                a0, a1, a2, a3 = accs
                base = g * L + i * 8
                for r in range(0, 8, 2):
                    a0 = a0 + rows_v[base + r, pl.ds(0, 16)]
                    a1 = a1 + rows_v[base + r, pl.ds(16, 16)]
                    a2 = a2 + rows_v[base + r + 1, pl.ds(0, 16)]
                    a3 = a3 + rows_v[base + r + 1, pl.ds(16, 16)]
                return a0, a1, a2, a3

            zero = jnp.zeros((16,), jnp.float32)
            a0, a1, a2, a3 = lax.fori_loop(
                0, L // 8, red_body, (zero, zero, zero, zero))
            out_v[g, pl.ds(0, 16)] = (a0 + a2) * INV_L
            out_v[g, pl.ds(16, 16)] = (a1 + a3) * INV_L
        pltpu.sync_copy(out_v, out_hbm.at[pl.ds(out_base, G)])

    start(0, 0)

    def pair_body(i, _):
        c0 = i * 2
        c1 = c0 + 1
        start(c1, 1)
        drain(0)
        reduce_store(c0, 0)

        @pl.when(c1 + 1 < CHUNKS)
        def _():
            start(c1 + 1, 0)

        drain(1)
        reduce_store(c1, 1)
        return ()

    lax.fori_loop(0, CHUNKS // 2, pair_body, ())


def kernel(state_tensor, table):
    if state_tensor.dtype != jnp.int32:
        state_tensor = state_tensor.astype(jnp.int32)
    tail = lax.slice(table, (NBLK * BLK, 0), (V, D)).reshape(16, BLK)
    tab_lin = _transpose_table(table.T, tail).reshape(VPAD, D)
    return _pooled_gather(tab_lin, state_tensor)


# speedup vs baseline: 1.1203x; 1.1203x over previous
import functools

import jax
import jax.numpy as jnp
from jax import lax
from jax.experimental import pallas as pl
from jax.experimental.pallas import tpu as pltpu
from jax.experimental.pallas import tpu_sc as plsc

B = 16384
L = 200
D = 32
NC = 2
NS = 16
NW = NC * NS
ROWS_PER_W = B // NW
G = 8
CHUNKS = ROWS_PER_W // G
IDX_SPLITS = ((0, 104), (104, 96))
INV_L = 1.0 / L

_mesh = plsc.VectorSubcoreMesh(core_axis_name="c", subcore_axis_name="s")


@functools.partial(
    pl.kernel,
    out_type=jax.ShapeDtypeStruct((B, D), jnp.float32),
    mesh=_mesh,
    compiler_params=pltpu.CompilerParams(use_tc_tiling_on_sc=False),
    scratch_types=[
        pltpu.VMEM((2, G, L), jnp.int32),
        pltpu.VMEM((G * L, D), jnp.float32),
        pltpu.VMEM((G * L, D), jnp.float32),
        pltpu.VMEM((G, D), jnp.float32),
        pltpu.SemaphoreType.DMA,
        pltpu.SemaphoreType.DMA,
    ],
)
def _pooled_gather(table_hbm, idx_hbm, out_hbm, idx_v, rows0_v, rows1_v,
                   out_v, sem0, sem1):
    wid = lax.axis_index("s") * NC + lax.axis_index("c")
    row_base = wid * ROWS_PER_W
    rows_bufs = (rows0_v, rows1_v)
    sems = (sem0, sem1)

    def start(c, buf):
        out_base = pl.multiple_of(row_base + c * G, G)
        pltpu.sync_copy(idx_hbm.at[pl.ds(out_base, G)], idx_v.at[buf])
        for g in range(G):
            for off, size in IDX_SPLITS:
                pltpu.async_copy(
                    table_hbm.at[idx_v.at[buf, g, pl.ds(off, size)]],
                    rows_bufs[buf].at[pl.ds(g * L + off, size)],
                    sems[buf],
                )

    def drain(buf):
        pltpu.make_async_copy(
            table_hbm.at[pl.ds(0, G * L)], rows_bufs[buf], sems[buf]
        ).wait()

    def reduce_store(c, buf):
        rows_v = rows_bufs[buf]
        out_base = pl.multiple_of(row_base + c * G, G)
        for g in range(G):
            def red_body(i, accs):
                a0, a1, a2, a3 = accs
                base = g * L + i * 8
                for r in range(0, 8, 2):
                    a0 = a0 + rows_v[base + r, pl.ds(0, 16)]
                    a1 = a1 + rows_v[base + r, pl.ds(16, 16)]
                    a2 = a2 + rows_v[base + r + 1, pl.ds(0, 16)]
                    a3 = a3 + rows_v[base + r + 1, pl.ds(16, 16)]
                return a0, a1, a2, a3

            zero = jnp.zeros((16,), jnp.float32)
            a0, a1, a2, a3 = lax.fori_loop(
                0, L // 8, red_body, (zero, zero, zero, zero))
            out_v[g, pl.ds(0, 16)] = (a0 + a2) * INV_L
            out_v[g, pl.ds(16, 16)] = (a1 + a3) * INV_L
        pltpu.sync_copy(out_v, out_hbm.at[pl.ds(out_base, G)])

    start(0, 0)

    def pair_body(i, _):
        c0 = i * 2
        c1 = c0 + 1
        start(c1, 1)
        drain(0)
        reduce_store(c0, 0)

        @pl.when(c1 + 1 < CHUNKS)
        def _():
            start(c1 + 1, 0)

        drain(1)
        reduce_store(c1, 1)
        return ()

    lax.fori_loop(0, CHUNKS // 2, pair_body, ())


def kernel(state_tensor, table):
    if state_tensor.dtype != jnp.int32:
        state_tensor = state_tensor.astype(jnp.int32)
    return _pooled_gather(table, state_tensor)
